# baseline (device time: 31764 ns/iter reference)
import jax
import jax.numpy as jnp
from jax import lax
from jax.experimental import pallas as pl
from jax.experimental.pallas import tpu as pltpu

N_DEV = 4
E_PER = 4
HALF = E_PER // 2


def kernel(x, router_W, route_idx, expert_W, shared_W):
    m, d = x.shape
    e_loc, _, h = expert_W.shape
    n_exp = router_W.shape[1]

    def body(x_ref, rw_ref, idx_ref, ew_ref, sw_ref, out_ref,
             myg, grpL, grpR, grpO, s1, s2, rL, rR, rO):
        my = lax.axis_index("i")
        left = (my - 1) % N_DEV
        right = (my + 1) % N_DEV

        barrier_sem = pltpu.get_barrier_semaphore()
        for nbr in (left, right):
            pl.semaphore_signal(
                barrier_sem, inc=1,
                device_id=(nbr,), device_id_type=pl.DeviceIdType.MESH,
            )
        pl.semaphore_wait(barrier_sem, 2)

        myg[...] = ew_ref[...].astype(jnp.bfloat16)

        p1_r = pltpu.make_async_remote_copy(
            src_ref=myg, dst_ref=grpL, send_sem=s1.at[0], recv_sem=rL,
            device_id=(right,), device_id_type=pl.DeviceIdType.MESH,
        )
        p1_l = pltpu.make_async_remote_copy(
            src_ref=myg, dst_ref=grpR, send_sem=s1.at[1], recv_sem=rR,
            device_id=(left,), device_id_type=pl.DeviceIdType.MESH,
        )
        p1_r.start()
        p1_l.start()

        xb = x_ref[...].astype(jnp.bfloat16)
        scores = jnp.dot(x_ref[...], rw_ref[...],
                         preferred_element_type=jnp.float32)
        s_max = jnp.max(scores, axis=1, keepdims=True)
        p = jnp.exp(scores - s_max)
        probs = p / jnp.sum(p, axis=1, keepdims=True)
        idx = idx_ref[...]
        lane = lax.broadcasted_iota(jnp.int32, (m, n_exp), 1)
        p_routed = jnp.sum(jnp.where(lane == idx, probs, 0.0),
                           axis=1, keepdims=True)

        def accum_group(w_group, origin, acc):
            origin, off = origin
            for j in range(w_group.shape[0]):
                gid = origin * E_PER + off + j
                coef = jnp.where(idx == gid, p_routed, 0.0)
                xm = xb * coef.astype(jnp.bfloat16)
                acc = acc + jnp.dot(xm, w_group[j],
                                    preferred_element_type=jnp.float32)
            return acc

        acc = jnp.dot(xb, sw_ref[...].astype(jnp.bfloat16),
                      preferred_element_type=jnp.float32)
        acc = accum_group(myg[...], (my, 0), acc)

        p1_r.wait_recv()
        p1_l.wait_recv()
        p2_l = pltpu.make_async_remote_copy(
            src_ref=grpR.at[pl.ds(0, HALF)], dst_ref=grpO.at[pl.ds(0, HALF)],
            send_sem=s2.at[0], recv_sem=rO.at[0],
            device_id=(left,), device_id_type=pl.DeviceIdType.MESH,
        )
        p2_r = pltpu.make_async_remote_copy(
            src_ref=grpL.at[pl.ds(HALF, HALF)],
            dst_ref=grpO.at[pl.ds(HALF, HALF)],
            send_sem=s2.at[1], recv_sem=rO.at[1],
            device_id=(right,), device_id_type=pl.DeviceIdType.MESH,
        )
        p2_l.start()
        p2_r.start()

        acc = accum_group(grpL[...], (left, 0), acc)
        acc = accum_group(grpR[...], (right, 0), acc)

        p1_r.wait_send()
        p1_l.wait_send()

        opp = (my + 2) % N_DEV
        p2_l.wait_recv()
        acc = accum_group(grpO[pl.ds(0, HALF)], (opp, 0), acc)
        p2_r.wait_recv()
        acc = accum_group(grpO[pl.ds(HALF, HALF)], (opp, HALF), acc)

        p2_l.wait_send()
        p2_r.wait_send()
        out_ref[...] = acc

    return pl.pallas_call(
        body,
        out_shape=jax.ShapeDtypeStruct((m, h), jnp.float32),
        in_specs=[pl.BlockSpec(memory_space=pltpu.VMEM)] * 5,
        out_specs=pl.BlockSpec(memory_space=pltpu.VMEM),
        scratch_shapes=[
            pltpu.VMEM((E_PER, d, h), jnp.bfloat16),
            pltpu.VMEM((E_PER, d, h), jnp.bfloat16),
            pltpu.VMEM((E_PER, d, h), jnp.bfloat16),
            pltpu.VMEM((E_PER, d, h), jnp.bfloat16),
            pltpu.SemaphoreType.DMA((2,)),
            pltpu.SemaphoreType.DMA((2,)),
            pltpu.SemaphoreType.DMA,
            pltpu.SemaphoreType.DMA,
            pltpu.SemaphoreType.DMA((2,)),
        ],
        compiler_params=pltpu.CompilerParams(collective_id=0),
    )(x, router_W, route_idx, expert_W, shared_W)


# device time: 30251 ns/iter; 1.0500x vs baseline; 1.0500x over previous
import os

import jax
import jax.numpy as jnp
from jax import lax
from jax.experimental import pallas as pl
from jax.experimental.pallas import tpu as pltpu

ABLATE = os.environ.get("ABLATE", "")

N_DEV = 4
E_PER = 4
HALF = E_PER // 2


def kernel(x, router_W, route_idx, expert_W, shared_W):
    m, d = x.shape
    e_loc, _, h = expert_W.shape
    n_exp = router_W.shape[1]

    def body(x_ref, rw_ref, idx_ref, ew_ref, sw_ref, out_ref,
             myg, grpL, grpR, grpO, s1, s2, rL, rR, rO):
        my = lax.axis_index("i")
        left = (my - 1) % N_DEV
        right = (my + 1) % N_DEV

        if ABLATE != "compute":
            barrier_sem = pltpu.get_barrier_semaphore()
            for nbr in (left, right):
                pl.semaphore_signal(
                    barrier_sem, inc=1,
                    device_id=(nbr,), device_id_type=pl.DeviceIdType.MESH,
                )
            pl.semaphore_wait(barrier_sem, 2)

        myg[...] = ew_ref[...].astype(jnp.bfloat16)

        if ABLATE != "compute":
            p1_r = pltpu.make_async_remote_copy(
                src_ref=myg, dst_ref=grpL, send_sem=s1.at[0], recv_sem=rL,
                device_id=(right,), device_id_type=pl.DeviceIdType.MESH,
            )
            p1_l = pltpu.make_async_remote_copy(
                src_ref=myg, dst_ref=grpR, send_sem=s1.at[1], recv_sem=rR,
                device_id=(left,), device_id_type=pl.DeviceIdType.MESH,
            )
            p1_r.start()
            p1_l.start()

        xb = x_ref[...].astype(jnp.bfloat16)
        scores = jnp.dot(x_ref[...], rw_ref[...],
                         preferred_element_type=jnp.float32)
        s_max = jnp.max(scores, axis=1, keepdims=True)
        p = jnp.exp(scores - s_max)
        probs = p / jnp.sum(p, axis=1, keepdims=True)
        idx = idx_ref[...]
        lane = lax.broadcasted_iota(jnp.int32, (m, n_exp), 1)
        p_routed = jnp.sum(jnp.where(lane == idx, probs, 0.0),
                           axis=1, keepdims=True)

        def accum_group(w_group, origin, acc):
            if ABLATE == "comm":
                return acc
            origin, off = origin
            for j in range(w_group.shape[0]):
                gid = origin * E_PER + off + j
                coef = jnp.where(idx == gid, p_routed, 0.0)
                xm = xb * coef.astype(jnp.bfloat16)
                acc = acc + jnp.dot(xm, w_group[j],
                                    preferred_element_type=jnp.float32)
            return acc

        if ABLATE == "comm":
            acc = jnp.zeros((m, h), jnp.float32)
        else:
            acc = jnp.dot(xb, sw_ref[...].astype(jnp.bfloat16),
                          preferred_element_type=jnp.float32)
        acc = accum_group(myg[...], (my, 0), acc)

        if ABLATE == "compute":
            for o in range(1, N_DEV):
                acc = accum_group(myg[...], ((my + o) % N_DEV, 0), acc)
            out_ref[...] = acc
            return

        p1_r.wait_recv()
        p1_l.wait_recv()
        p2_l = pltpu.make_async_remote_copy(
            src_ref=grpR.at[pl.ds(0, HALF)], dst_ref=grpO.at[pl.ds(0, HALF)],
            send_sem=s2.at[0], recv_sem=rO.at[0],
            device_id=(left,), device_id_type=pl.DeviceIdType.MESH,
        )
        p2_r = pltpu.make_async_remote_copy(
            src_ref=grpL.at[pl.ds(HALF, HALF)],
            dst_ref=grpO.at[pl.ds(HALF, HALF)],
            send_sem=s2.at[1], recv_sem=rO.at[1],
            device_id=(right,), device_id_type=pl.DeviceIdType.MESH,
        )
        p2_l.start()
        p2_r.start()

        acc = accum_group(grpL[...], (left, 0), acc)
        acc = accum_group(grpR[...], (right, 0), acc)

        p1_r.wait_send()
        p1_l.wait_send()

        opp = (my + 2) % N_DEV
        p2_l.wait_recv()
        acc = accum_group(grpO[pl.ds(0, HALF)], (opp, 0), acc)
        p2_r.wait_recv()
        acc = accum_group(grpO[pl.ds(HALF, HALF)], (opp, HALF), acc)

        p2_l.wait_send()
        p2_r.wait_send()
        out_ref[...] = acc

    return pl.pallas_call(
        body,
        out_shape=jax.ShapeDtypeStruct((m, h), jnp.float32),
        in_specs=[pl.BlockSpec(memory_space=pltpu.VMEM)] * 5,
        out_specs=pl.BlockSpec(memory_space=pltpu.VMEM),
        scratch_shapes=[
            pltpu.VMEM((E_PER, d, h), jnp.bfloat16),
            pltpu.VMEM((E_PER, d, h), jnp.bfloat16),
            pltpu.VMEM((E_PER, d, h), jnp.bfloat16),
            pltpu.VMEM((E_PER, d, h), jnp.bfloat16),
            pltpu.SemaphoreType.DMA((2,)),
            pltpu.SemaphoreType.DMA((2,)),
            pltpu.SemaphoreType.DMA,
            pltpu.SemaphoreType.DMA,
            pltpu.SemaphoreType.DMA((2,)),
        ],
        compiler_params=pltpu.CompilerParams(collective_id=0),
    )(x, router_W, route_idx, expert_W, shared_W)


# device time: 14018 ns/iter; 2.2659x vs baseline; 2.1580x over previous
import os

import jax
import jax.numpy as jnp
from jax import lax
from jax.experimental import pallas as pl
from jax.experimental.pallas import tpu as pltpu

ABLATE = os.environ.get("ABLATE", "")

N_DEV = 4
E_PER = 4
HALF = E_PER // 2


def kernel(x, router_W, route_idx, expert_W, shared_W):
    m, d = x.shape
    e_loc, _, h = expert_W.shape
    n_exp = router_W.shape[1]

    def body(x_ref, rw_ref, idx_ref, ew_ref, sw_ref, out_ref,
             myg, grpL, grpR, grpO, s1, s2, rL, rR, rO):
        my = lax.axis_index("i")
        left = (my - 1) % N_DEV
        right = (my + 1) % N_DEV

        if ABLATE != "compute":
            barrier_sem = pltpu.get_barrier_semaphore()
            for nbr in (left, right):
                pl.semaphore_signal(
                    barrier_sem, inc=1,
                    device_id=(nbr,), device_id_type=pl.DeviceIdType.MESH,
                )
            pl.semaphore_wait(barrier_sem, 2)

        myg[...] = ew_ref[...].astype(jnp.bfloat16)

        if ABLATE != "compute":
            p1_r = pltpu.make_async_remote_copy(
                src_ref=myg, dst_ref=grpL, send_sem=s1.at[0], recv_sem=rL,
                device_id=(right,), device_id_type=pl.DeviceIdType.MESH,
            )
            p1_l = pltpu.make_async_remote_copy(
                src_ref=myg, dst_ref=grpR, send_sem=s1.at[1], recv_sem=rR,
                device_id=(left,), device_id_type=pl.DeviceIdType.MESH,
            )
            p1_r.start()
            p1_l.start()

        xb = x_ref[...].astype(jnp.bfloat16)
        scores = jnp.dot(x_ref[...], rw_ref[...],
                         preferred_element_type=jnp.float32)
        s_max = jnp.max(scores, axis=1, keepdims=True)
        p = jnp.exp(scores - s_max)
        probs = p / jnp.sum(p, axis=1, keepdims=True)
        idx = idx_ref[...]
        lane = lax.broadcasted_iota(jnp.int32, (m, n_exp), 1)
        p_routed = jnp.sum(jnp.where(lane == idx, probs, 0.0),
                           axis=1, keepdims=True)

        def accum_group(w_group, origin, acc):
            if ABLATE == "comm":
                return acc
            origin, off = origin
            for j in range(w_group.shape[0]):
                gid = origin * E_PER + off + j
                coef = jnp.where(idx == gid, p_routed, 0.0)
                xm = xb * coef.astype(jnp.bfloat16)
                acc = acc + jnp.dot(xm, w_group[j],
                                    preferred_element_type=jnp.float32)
            return acc

        if ABLATE == "comm":
            acc = jnp.zeros((m, h), jnp.float32)
        else:
            acc = jnp.dot(xb, sw_ref[...].astype(jnp.bfloat16),
                          preferred_element_type=jnp.float32)
        acc = accum_group(myg[...], (my, 0), acc)

        if ABLATE == "compute":
            for o in range(1, N_DEV):
                acc = accum_group(myg[...], ((my + o) % N_DEV, 0), acc)
            out_ref[...] = acc
            return

        p1_r.wait_recv()
        p1_l.wait_recv()
        p2_l = pltpu.make_async_remote_copy(
            src_ref=grpR.at[pl.ds(0, HALF)], dst_ref=grpO.at[pl.ds(0, HALF)],
            send_sem=s2.at[0], recv_sem=rO.at[0],
            device_id=(left,), device_id_type=pl.DeviceIdType.MESH,
        )
        p2_r = pltpu.make_async_remote_copy(
            src_ref=grpL.at[pl.ds(HALF, HALF)],
            dst_ref=grpO.at[pl.ds(HALF, HALF)],
            send_sem=s2.at[1], recv_sem=rO.at[1],
            device_id=(right,), device_id_type=pl.DeviceIdType.MESH,
        )
        p2_l.start()
        p2_r.start()

        acc = accum_group(grpL[...], (left, 0), acc)
        acc = accum_group(grpR[...], (right, 0), acc)

        p1_r.wait_send()
        p1_l.wait_send()

        opp = (my + 2) % N_DEV
        p2_l.wait_recv()
        acc = accum_group(grpO[pl.ds(0, HALF)], (opp, 0), acc)
        p2_r.wait_recv()
        acc = accum_group(grpO[pl.ds(HALF, HALF)], (opp, HALF), acc)

        p2_l.wait_send()
        p2_r.wait_send()
        out_ref[...] = acc

    return pl.pallas_call(
        body,
        out_shape=jax.ShapeDtypeStruct((m, h), jnp.float32),
        in_specs=[pl.BlockSpec(memory_space=pltpu.VMEM)] * 5,
        out_specs=pl.BlockSpec(memory_space=pltpu.VMEM),
        scratch_shapes=[
            pltpu.VMEM((E_PER, d, h), jnp.bfloat16),
            pltpu.VMEM((E_PER, d, h), jnp.bfloat16),
            pltpu.VMEM((E_PER, d, h), jnp.bfloat16),
            pltpu.VMEM((E_PER, d, h), jnp.bfloat16),
            pltpu.SemaphoreType.DMA((2,)),
            pltpu.SemaphoreType.DMA((2,)),
            pltpu.SemaphoreType.DMA,
            pltpu.SemaphoreType.DMA,
            pltpu.SemaphoreType.DMA((2,)),
        ],
        compiler_params=(
            pltpu.CompilerParams()
            if ABLATE == "compute"
            else pltpu.CompilerParams(collective_id=0)
        ),
    )(x, router_W, route_idx, expert_W, shared_W)
